# baseline (speedup 1.0000x reference)
import jax
import jax.numpy as jnp
from jax import lax
from jax.experimental import pallas as pl
from jax.experimental.pallas import tpu as pltpu
from jax.experimental.pallas import tpu_sc as plsc

NUM_VIEWS = 16
TOKEN_DIM = 256
BATCH = 16384
NUM_CORES = 2
NUM_SUBCORES = 16
NUM_WORKERS = NUM_CORES * NUM_SUBCORES
ROWS_PER_WORKER = BATCH // NUM_WORKERS
COPIES_PER_WORKER = 8
CHUNK = 128
NUM_CHUNKS = ROWS_PER_WORKER // CHUNK
NBUF = 3


def _gather_body(idx_hbm, table_hbm, out_hbm, idx_v, *bufs_and_sems):
    bufs = bufs_and_sems[:NBUF]
    gsems = bufs_and_sems[NBUF:NBUF + NUM_CHUNKS]
    wsems = bufs_and_sems[NBUF + NUM_CHUNKS:]
    wid = lax.axis_index("s") * NUM_CORES + lax.axis_index("c")
    base = wid * ROWS_PER_WORKER

    pltpu.sync_copy(idx_hbm.at[wid], idx_v)

    gathers = [None] * NUM_CHUNKS
    writes = [None] * NUM_CHUNKS

    def start_gather(j):
        gathers[j] = pltpu.async_copy(
            table_hbm.at[idx_v.at[j]], bufs[j % NBUF], gsems[j])

    for j in range(NBUF):
        start_gather(j)
    for j in range(NUM_CHUNKS):
        gathers[j].wait()
        writes[j] = pltpu.async_copy(
            bufs[j % NBUF], out_hbm.at[pl.ds(base + j * CHUNK, CHUNK)],
            wsems[j])
        nxt = j + NBUF
        if nxt < NUM_CHUNKS:
            writes[nxt - NBUF].wait()
            start_gather(nxt)
    for j in range(max(0, NUM_CHUNKS - NBUF), NUM_CHUNKS):
        writes[j].wait()


@jax.jit
def kernel(view_id, view_embed):
    idx = view_id.astype(jnp.int32).reshape(NUM_WORKERS, ROWS_PER_WORKER)
    table_rep = jnp.tile(view_embed, (NUM_WORKERS * COPIES_PER_WORKER, 1))
    copy_id = (jnp.arange(NUM_WORKERS, dtype=jnp.int32)[:, None]
               * COPIES_PER_WORKER
               + jnp.arange(ROWS_PER_WORKER, dtype=jnp.int32)[None, :]
               % COPIES_PER_WORKER)
    idx = (idx + copy_id * NUM_VIEWS).reshape(NUM_WORKERS, NUM_CHUNKS, CHUNK)
    run = pl.kernel(
        _gather_body,
        out_type=jax.ShapeDtypeStruct((BATCH, TOKEN_DIM), jnp.float32),
        mesh=plsc.VectorSubcoreMesh(core_axis_name="c", subcore_axis_name="s"),
        scratch_types=(
            [pltpu.VMEM((NUM_CHUNKS, CHUNK), jnp.int32)]
            + [pltpu.VMEM((CHUNK, TOKEN_DIM), jnp.float32)] * NBUF
            + [pltpu.SemaphoreType.DMA] * (2 * NUM_CHUNKS)
        ),
    )
    return run(idx, table_rep)


# speedup vs baseline: 1.0239x; 1.0239x over previous
import jax
import jax.numpy as jnp
from jax import lax
from jax.experimental import pallas as pl
from jax.experimental.pallas import tpu as pltpu
from jax.experimental.pallas import tpu_sc as plsc

NUM_VIEWS = 16
TOKEN_DIM = 256
BATCH = 16384
NUM_CORES = 2
NUM_SUBCORES = 16
NUM_WORKERS = NUM_CORES * NUM_SUBCORES
ROWS_PER_WORKER = BATCH // NUM_WORKERS
COPIES_PER_WORKER = 8
CHUNK = 64
NUM_CHUNKS = ROWS_PER_WORKER // CHUNK
NBUF = 7


def _gather_body(idx_hbm, table_hbm, out_hbm, idx_v, *bufs_and_sems):
    bufs = bufs_and_sems[:NBUF]
    gsems = bufs_and_sems[NBUF:NBUF + NUM_CHUNKS]
    wsems = bufs_and_sems[NBUF + NUM_CHUNKS:]
    wid = lax.axis_index("s") * NUM_CORES + lax.axis_index("c")
    base = wid * ROWS_PER_WORKER

    pltpu.sync_copy(idx_hbm.at[wid], idx_v)

    gathers = [None] * NUM_CHUNKS
    writes = [None] * NUM_CHUNKS

    def start_gather(j):
        gathers[j] = pltpu.async_copy(
            table_hbm.at[idx_v.at[j]], bufs[j % NBUF], gsems[j])

    for j in range(NBUF):
        start_gather(j)
    for j in range(NUM_CHUNKS):
        gathers[j].wait()
        writes[j] = pltpu.async_copy(
            bufs[j % NBUF], out_hbm.at[pl.ds(base + j * CHUNK, CHUNK)],
            wsems[j])
        nxt = j + NBUF
        if nxt < NUM_CHUNKS:
            writes[nxt - NBUF].wait()
            start_gather(nxt)
    for j in range(max(0, NUM_CHUNKS - NBUF), NUM_CHUNKS):
        writes[j].wait()


@jax.jit
def kernel(view_id, view_embed):
    idx = view_id.astype(jnp.int32).reshape(NUM_WORKERS, ROWS_PER_WORKER)
    table_rep = jnp.tile(view_embed, (NUM_WORKERS * COPIES_PER_WORKER, 1))
    copy_id = (jnp.arange(NUM_WORKERS, dtype=jnp.int32)[:, None]
               * COPIES_PER_WORKER
               + jnp.arange(ROWS_PER_WORKER, dtype=jnp.int32)[None, :]
               % COPIES_PER_WORKER)
    idx = (idx + copy_id * NUM_VIEWS).reshape(NUM_WORKERS, NUM_CHUNKS, CHUNK)
    run = pl.kernel(
        _gather_body,
        out_type=jax.ShapeDtypeStruct((BATCH, TOKEN_DIM), jnp.float32),
        mesh=plsc.VectorSubcoreMesh(core_axis_name="c", subcore_axis_name="s"),
        scratch_types=(
            [pltpu.VMEM((NUM_CHUNKS, CHUNK), jnp.int32)]
            + [pltpu.VMEM((CHUNK, TOKEN_DIM), jnp.float32)] * NBUF
            + [pltpu.SemaphoreType.DMA] * (2 * NUM_CHUNKS)
        ),
    )
    return run(idx, table_rep)
